# baseline (speedup 1.0000x reference)
import functools

import jax
import jax.numpy as jnp
from jax.experimental import pallas as pl
from jax.experimental.pallas import tpu as pltpu

_BS = 64


def _bias_kernel(row_x_ref, row_y_ref, col_x_ref, col_y_ref, tab_ref,
                 out_ref, scratch_ref, sem):
    i = pl.program_id(0)
    n = pl.num_programs(0)
    b_sz = out_ref.shape[0]
    bs = scratch_ref.shape[1]
    slot = jax.lax.rem(i, 2)

    def wait_slot(s, step):
        for b in range(b_sz):
            pltpu.make_async_copy(
                scratch_ref.at[s],
                out_ref.at[b, pl.ds(step * bs, bs)],
                sem.at[s, b],
            ).wait()

    @pl.when(i >= 2)
    def _():
        wait_slot(slot, i - 2)

    rx = row_x_ref[...][:, :, None]
    ry = row_y_ref[...][:, :, None]
    cx = col_x_ref[...][None, :, :]
    cy = col_y_ref[...][None, :, :]
    dist = jnp.abs(rx - cx) + jnp.abs(ry - cy)
    dist = jnp.minimum(dist, 8.0)
    acc = jnp.broadcast_to(tab_ref[0, :][None, None, :], dist.shape)
    for d in range(1, 9):
        acc = jnp.where(dist == float(d), tab_ref[d, :][None, None, :], acc)
    scratch_ref[slot] = acc

    for b in range(b_sz):
        pltpu.make_async_copy(
            scratch_ref.at[slot],
            out_ref.at[b, pl.ds(i * bs, bs)],
            sem.at[slot, b],
        ).start()

    @pl.when(i == n - 1)
    def _():
        wait_slot(1 - slot, i - 1)
        wait_slot(slot, i)


@functools.partial(jax.jit, static_argnums=(3,))
def _run(stab_xy, syndrome, dist_emb, S):
    B = syndrome.shape[0]
    DB = dist_emb.shape[1]
    xy = stab_xy.astype(jnp.float32)
    row_x = xy[:, 0:1]
    row_y = xy[:, 1:2]
    col_of_lane = jnp.arange(128, dtype=jnp.int32) // DB
    col_idx = 8 * jnp.arange(S // 8, dtype=jnp.int32)[:, None] + col_of_lane[None, :]
    col_x = xy[col_idx, 0]
    col_y = xy[col_idx, 1]
    tab = jnp.tile(dist_emb, (1, 128 // DB))

    grid = (S // _BS,)
    out = pl.pallas_call(
        _bias_kernel,
        grid=grid,
        in_specs=[
            pl.BlockSpec((_BS, 1), lambda i: (i, 0)),
            pl.BlockSpec((_BS, 1), lambda i: (i, 0)),
            pl.BlockSpec((S // 8, 128), lambda i: (0, 0)),
            pl.BlockSpec((S // 8, 128), lambda i: (0, 0)),
            pl.BlockSpec((9, 128), lambda i: (0, 0)),
        ],
        out_specs=pl.BlockSpec(memory_space=pl.ANY),
        out_shape=jax.ShapeDtypeStruct((B, S, S // 8, 128), jnp.float32),
        scratch_shapes=[
            pltpu.VMEM((2, _BS, S // 8, 128), jnp.float32),
            pltpu.SemaphoreType.DMA((2, B)),
        ],
    )(row_x, row_y, col_x, col_y, tab)
    return out.reshape(B, S, S, DB)


def kernel(stab_xy, syndrome, dist_emb, S):
    return _run(stab_xy, syndrome, dist_emb, stab_xy.shape[0])


# speedup vs baseline: 1.2246x; 1.2246x over previous
import functools

import jax
import jax.numpy as jnp
from jax.experimental import pallas as pl
from jax.experimental.pallas import tpu as pltpu

_BS = 64


def _bias_kernel(row_x_ref, row_y_ref, col_x_ref, col_y_ref, tab_ref,
                 out_ref, scratch_ref, sem):
    i = pl.program_id(0)
    n = pl.num_programs(0)
    b_sz = out_ref.shape[0]
    bs = scratch_ref.shape[1]
    slot = jax.lax.rem(i, 2)

    def wait_slot(s, step):
        for b in range(b_sz):
            pltpu.make_async_copy(
                scratch_ref.at[s],
                out_ref.at[b, pl.ds(step * bs, bs)],
                sem.at[s, b],
            ).wait()

    @pl.when(i >= 2)
    def _():
        wait_slot(slot, i - 2)

    rx = row_x_ref[...][:, :, None]
    ry = row_y_ref[...][:, :, None]
    cx = col_x_ref[...][None, :, :]
    cy = col_y_ref[...][None, :, :]
    dist = jnp.abs(rx - cx) + jnp.abs(ry - cy)
    dist = jnp.minimum(dist, 8.0)
    acc = jnp.broadcast_to(tab_ref[0, :][None, None, :], dist.shape)
    for d in range(1, 9):
        acc = jnp.where(dist == float(d), tab_ref[d, :][None, None, :], acc)
    scratch_ref[slot] = acc

    for b in range(b_sz):
        pltpu.make_async_copy(
            scratch_ref.at[slot],
            out_ref.at[b, pl.ds(i * bs, bs)],
            sem.at[slot, b],
        ).start()

    @pl.when(i == n - 1)
    def _():
        wait_slot(1 - slot, i - 1)
        wait_slot(slot, i)


def _run(stab_xy, syndrome, dist_emb, S):
    B = syndrome.shape[0]
    DB = dist_emb.shape[1]
    xy = stab_xy.astype(jnp.float32)
    row_x = xy[:, 0:1]
    row_y = xy[:, 1:2]
    col_of_lane = jnp.arange(128, dtype=jnp.int32) // DB
    col_idx = 8 * jnp.arange(S // 8, dtype=jnp.int32)[:, None] + col_of_lane[None, :]
    col_x = xy[col_idx, 0]
    col_y = xy[col_idx, 1]
    tab = jnp.tile(dist_emb, (1, 128 // DB))

    grid = (S // _BS,)
    out = pl.pallas_call(
        _bias_kernel,
        grid=grid,
        in_specs=[
            pl.BlockSpec((_BS, 1), lambda i: (i, 0)),
            pl.BlockSpec((_BS, 1), lambda i: (i, 0)),
            pl.BlockSpec((S // 8, 128), lambda i: (0, 0)),
            pl.BlockSpec((S // 8, 128), lambda i: (0, 0)),
            pl.BlockSpec((9, 128), lambda i: (0, 0)),
        ],
        out_specs=pl.BlockSpec(memory_space=pl.ANY),
        out_shape=jax.ShapeDtypeStruct((B, S, S // 8, 128), jnp.float32),
        scratch_shapes=[
            pltpu.VMEM((2, _BS, S // 8, 128), jnp.float32),
            pltpu.SemaphoreType.DMA((2, B)),
        ],
    )(row_x, row_y, col_x, col_y, tab)
    return out.reshape(B, S, S, DB)


def kernel(stab_xy, syndrome, dist_emb, S):
    B = syndrome.shape[0]
    s_static = stab_xy.shape[0]
    devs = jax.devices()
    nd = len(devs)
    while nd > 1 and B % nd != 0:
        nd -= 1
    if nd <= 1:
        return _run(stab_xy, syndrome, dist_emb, s_static)
    mesh = jax.sharding.Mesh(devs[:nd], ("b",))
    P = jax.sharding.PartitionSpec
    f = jax.shard_map(
        lambda xy, syn, emb: _run(xy, syn, emb, s_static),
        mesh=mesh,
        in_specs=(P(), P("b"), P()),
        out_specs=P("b"),
        check_vma=False,
    )
    return f(stab_xy, syndrome, dist_emb)
